# baseline (speedup 1.0000x reference)
import functools

import jax
import jax.numpy as jnp
from jax import lax
from jax.experimental import pallas as pl
from jax.experimental.pallas import tpu as pltpu
from jax.experimental.pallas import tpu_sc as plsc

HID = 128
EDG = 320000
NC, NS, LANES = 2, 16, 16
NW = NC * NS
K = 80

_BLK = {2560: (512, 5), 5120: (512, 10), 10112: (632, 16)}


def _padded(n):
    return -(-(n + 1) // 128) * 128


def _silu(x):
    return x * jax.nn.sigmoid(x)


def _ln_rows(x, g, b):
    m = jnp.mean(x, axis=-1, keepdims=True)
    v = jnp.mean((x - m) ** 2, axis=-1, keepdims=True)
    return (x - m) / jnp.sqrt(v + 1e-5) * g + b


def _dt(x, w):
    return lax.dot_general(x, w, (((1,), (1,)), ((), ())),
                           preferred_element_type=jnp.float32)


def _zero_my_slice(acc, zblk, r0, rpt):
    zv = jnp.zeros((LANES,), jnp.float32)
    for r in range(8):
        for j in range(HID // LANES):
            zblk[r, pl.ds(j * LANES, LANES)] = zv
    for i in range(rpt // 8):
        pltpu.sync_copy(zblk, acc.at[pl.ds(r0 + 8 * i, 8)])


@functools.lru_cache(maxsize=None)
def _spmm_kernel(n):
    n_pad = _padded(n)
    rpt = n_pad // NS
    ept = EDG // NW
    nch = ept // K
    masked = n < 10000
    mesh = plsc.VectorSubcoreMesh(core_axis_name="c", subcore_axis_name="s")

    @functools.partial(
        pl.kernel,
        out_type=jax.ShapeDtypeStruct((NC, n_pad, HID), jnp.float32),
        mesh=mesh,
        scratch_types=[
            pltpu.VMEM_SHARED((n_pad, HID), jnp.float32),
            pltpu.VMEM((8, HID), jnp.float32),
            pltpu.VMEM((ept,), jnp.int32),
            pltpu.VMEM((ept,), jnp.int32),
            pltpu.VMEM((K,), jnp.int32),
            pltpu.VMEM((K,), jnp.int32),
            pltpu.VMEM((K, HID), jnp.float32),
            pltpu.SemaphoreType.DMA,
        ],
    )
    def spmm(dst_hbm, src_hbm, h_hbm, out_hbm, acc, zblk, dall, sall, gidx,
             sidx, rows, sem):
        cid = lax.axis_index("c")
        sid = lax.axis_index("s")
        wid = sid * NC + cid
        r0 = sid * rpt
        _zero_my_slice(acc, zblk, r0, rpt)
        e0 = wid * ept
        pltpu.sync_copy(dst_hbm.at[pl.ds(e0, ept)], dall)
        pltpu.sync_copy(src_hbm.at[pl.ds(e0, ept)], sall)
        plsc.subcore_barrier()

        def chunk(c, carry):
            base = c * K
            for j in range(K // LANES):
                vd = dall[pl.ds(base + j * LANES, LANES)]
                vs = sall[pl.ds(base + j * LANES, LANES)]
                if masked:
                    ok = (vd < n) & (vs < n)
                    vs = jnp.where(ok, vs, 0)
                    vd = jnp.where(ok, vd, n)
                gidx[pl.ds(j * LANES, LANES)] = vs
                sidx[pl.ds(j * LANES, LANES)] = vd
            pltpu.async_copy(h_hbm.at[gidx], rows, sem).wait()
            pltpu.sync_copy(rows, acc.at[sidx], add=True)
            return carry

        lax.fori_loop(0, nch, chunk, 0)
        plsc.subcore_barrier()
        pltpu.sync_copy(acc.at[pl.ds(r0, rpt)],
                        out_hbm.at[cid, pl.ds(r0, rpt)])

    return spmm


@functools.lru_cache(maxsize=None)
def _unpool_kernel(m_pad, n):
    n_pad = _padded(n)
    rpt = n_pad // NS
    nf = m_pad // K
    kmax = -(-nf // NW)
    mesh = plsc.VectorSubcoreMesh(core_axis_name="c", subcore_axis_name="s")

    scratch = [
        pltpu.VMEM_SHARED((n_pad, HID), jnp.float32),
        pltpu.VMEM((8, HID), jnp.float32),
        pltpu.VMEM((K,), jnp.int32),
        pltpu.VMEM((K, HID), jnp.float32),
    ]

    @functools.partial(
        pl.kernel,
        out_type=jax.ShapeDtypeStruct((NC, n_pad, HID), jnp.float32),
        mesh=mesh,
        scratch_types=scratch,
    )
    def unpool(rows_hbm, idx_hbm, out_hbm, acc, zblk, ibuf, rbuf):
        cid = lax.axis_index("c")
        sid = lax.axis_index("s")
        wid = sid * NC + cid
        r0 = sid * rpt
        _zero_my_slice(acc, zblk, r0, rpt)
        plsc.subcore_barrier()
        for k in range(kmax):
            c = wid + NW * k

            @pl.when(c < nf)
            def _():
                pltpu.sync_copy(idx_hbm.at[pl.ds(c * K, K)], ibuf)
                pltpu.sync_copy(rows_hbm.at[pl.ds(c * K, K)], rbuf)
                pltpu.sync_copy(rbuf, acc.at[ibuf], add=True)

        plsc.subcore_barrier()
        pltpu.sync_copy(acc.at[pl.ds(r0, rpt)],
                        out_hbm.at[cid, pl.ds(r0, rpt)])

    return unpool


def _lat1(z, w, b):
    def body(z_ref, w_ref, b_ref, o_ref):
        r = _silu(_dt(z_ref[...], w_ref[...]) + b_ref[...])
        o_ref[...] = jnp.broadcast_to(r, (8, 2 * HID))

    return pl.pallas_call(
        body, out_shape=jax.ShapeDtypeStruct((8, 2 * HID), jnp.float32),
    )(z, w, b)


def _lat2(w, h1, b):
    BR, G = 2000, 80

    def body(w_ref, h_ref, b_ref, o_ref):
        o_ref[...] = _dt(w_ref[...], h_ref[...]) + b_ref[...]

    return pl.pallas_call(
        body,
        grid=(G,),
        in_specs=[
            pl.BlockSpec((BR, 2 * HID), lambda i: (i, 0)),
            pl.BlockSpec((8, 2 * HID), lambda i: (0, 0)),
            pl.BlockSpec((BR, 8), lambda i: (i, 0)),
        ],
        out_specs=pl.BlockSpec((BR, 8), lambda i: (i, 0)),
        out_shape=jax.ShapeDtypeStruct((BR * G, 8), jnp.float32),
    )(w, h1, b)


def _up_ln(n_pad, pp, w, b, g, bl):
    BR, G = _BLK[n_pad]

    def body(p_ref, w_ref, b_ref, g_ref, bl_ref, h_ref, l_ref):
        a = p_ref[...]
        h = _dt(a[0] + a[1], w_ref[...]) + b_ref[...]
        h_ref[...] = h
        l_ref[...] = _ln_rows(h, g_ref[...], bl_ref[...])

    wspec = pl.BlockSpec((HID, HID), lambda i: (0, 0))
    vspec = pl.BlockSpec((1, HID), lambda i: (0, 0))
    rspec = pl.BlockSpec((BR, HID), lambda i: (i, 0))
    return pl.pallas_call(
        body,
        grid=(G,),
        in_specs=[pl.BlockSpec((NC, BR, HID), lambda i: (0, i, 0)),
                  wspec, vspec, vspec, vspec],
        out_specs=[rspec, rspec],
        out_shape=[jax.ShapeDtypeStruct((n_pad, HID), jnp.float32)] * 2,
    )(pp, w, b, g, bl)


def _combine(n_pad, h, hln, pp, eps, w1, b1, w2, b2, g=None, bl=None):
    BR, G = _BLK[n_pad]
    out_ln = g is not None

    def body(h_ref, l_ref, p_ref, e_ref, w1_ref, b1_ref, w2_ref, b2_ref,
             *rest):
        a = p_ref[...]
        x = (1.0 + e_ref[0, 0]) * l_ref[...] + a[0] + a[1]
        t = _silu(_dt(x, w1_ref[...]) + b1_ref[...])
        t = _dt(t, w2_ref[...]) + b2_ref[...]
        ho = h_ref[...] + t
        if out_ln:
            g_ref, bl_ref, ho_ref, lo_ref = rest
            ho_ref[...] = ho
            lo_ref[...] = _ln_rows(ho, g_ref[...], bl_ref[...])
        else:
            rest[0][...] = ho

    wspec = pl.BlockSpec((HID, HID), lambda i: (0, 0))
    vspec = pl.BlockSpec((1, HID), lambda i: (0, 0))
    rspec = pl.BlockSpec((BR, HID), lambda i: (i, 0))
    espec = pl.BlockSpec((1, 1), lambda i: (0, 0))
    in_specs = [rspec, rspec,
                pl.BlockSpec((NC, BR, HID), lambda i: (0, i, 0)),
                espec, wspec, vspec, wspec, vspec]
    args = [h, hln, pp, eps, w1, b1, w2, b2]
    if out_ln:
        in_specs += [vspec, vspec]
        args += [g, bl]
        out_specs = [rspec, rspec]
        out_shape = [jax.ShapeDtypeStruct((n_pad, HID), jnp.float32)] * 2
    else:
        out_specs = [rspec]
        out_shape = [jax.ShapeDtypeStruct((n_pad, HID), jnp.float32)]
    res = pl.pallas_call(body, grid=(G,), in_specs=in_specs,
                         out_specs=out_specs, out_shape=out_shape)(*args)
    return res if out_ln else (res[0], None)


def _pos_stats(pos):
    def body(p_ref, o_ref):
        x = p_ref[...]
        mean = jnp.sum(x, axis=0, keepdims=True) / 10000.0
        ss = jnp.sum((x - mean) ** 2, axis=0, keepdims=True)
        sd = jnp.sqrt(ss / 9999.0)
        o_ref[...] = jnp.concatenate([mean, sd], axis=0)

    return pl.pallas_call(
        body, out_shape=jax.ShapeDtypeStruct((2, 3), jnp.float32))(pos)


def _pe_add_ln(n_pad, pos, st, w1, b1, w2, b2, h, g, bl):
    BR, G = _BLK[n_pad]

    def body(p_ref, st_ref, w1_ref, b1_ref, w2_ref, b2_ref, h_ref, g_ref,
             bl_ref, ho_ref, lo_ref):
        s = st_ref[...]
        pn = (p_ref[...] - s[0:1, :]) / (s[1:2, :] + 1e-8)
        t = _silu(_dt(pn, w1_ref[...]) + b1_ref[...])
        pe = _dt(t, w2_ref[...]) + b2_ref[...]
        ho = h_ref[...] + pe
        ho_ref[...] = ho
        lo_ref[...] = _ln_rows(ho, g_ref[...], bl_ref[...])

    vspec = pl.BlockSpec((1, HID), lambda i: (0, 0))
    rspec = pl.BlockSpec((BR, HID), lambda i: (i, 0))
    return pl.pallas_call(
        body,
        grid=(G,),
        in_specs=[pl.BlockSpec((BR, 3), lambda i: (i, 0)),
                  pl.BlockSpec((2, 3), lambda i: (0, 0)),
                  pl.BlockSpec((HID, 3), lambda i: (0, 0)), vspec,
                  pl.BlockSpec((HID, HID), lambda i: (0, 0)), vspec,
                  rspec, vspec, vspec],
        out_specs=[rspec, rspec],
        out_shape=[jax.ShapeDtypeStruct((n_pad, HID), jnp.float32)] * 2,
    )(pos, st, w1, b1, w2, b2, h, g, bl)


def _final(h, g, bl, w, b):
    BR, G = 1000, 10

    def body(h_ref, g_ref, bl_ref, w_ref, b_ref, o_ref):
        x = _silu(_ln_rows(h_ref[...], g_ref[...], bl_ref[...]))
        o_ref[...] = _dt(x, w_ref[...]) + b_ref[...]

    vspec = pl.BlockSpec((1, HID), lambda i: (0, 0))
    rspec = pl.BlockSpec((BR, HID), lambda i: (i, 0))
    return pl.pallas_call(
        body,
        grid=(G,),
        in_specs=[rspec, vspec, vspec,
                  pl.BlockSpec((HID, HID), lambda i: (0, 0)), vspec],
        out_specs=rspec,
        out_shape=jax.ShapeDtypeStruct((BR * G, HID), jnp.float32),
    )(h, g, bl, w, b)


def kernel(z, pos, edge_index, keep_idx0, keep_idx1, keep_idx2, params):
    p = params
    r1 = lambda a: a.reshape(1, -1)
    dst_a, src_a = edge_index[0], edge_index[1]

    h1 = _lat1(z, p['lp1_W'], r1(p['lp1_b']))
    b2 = jnp.broadcast_to(p['lp2_b'].reshape(-1, 1), (160000, 8))
    h = _lat2(p['lp2_W'], h1, b2)[:, 0].reshape(1250, HID)
    h = jnp.pad(h, ((0, 30), (0, 0)))

    keeps = [keep_idx2, keep_idx1, keep_idx0]
    ms = [1250, 2500, 5000]
    ns = [2500, 5000, 10000]
    for d in range(3):
        m, n = ms[d], ns[d]
        n_pad = _padded(n)
        m_pad = -(-m // K) * K
        kp = jnp.concatenate(
            [keeps[d], jnp.full((m_pad - m,), n, jnp.int32)]) \
            if m_pad > m else keeps[d]
        pp = _unpool_kernel(m_pad, n)(h, kp)
        bp0, bp1 = p['stages'][d]
        h, hln = _up_ln(n_pad, pp, p['up_W'][d], r1(p['up_b'][d]),
                        r1(bp0['g']), r1(bp0['b']))
        for bi, bp in enumerate((bp0, bp1)):
            nbp = _spmm_kernel(n)(dst_a, src_a, hln)
            nxt = (r1(bp1['g']), r1(bp1['b'])) if bi == 0 else (None, None)
            h, hln = _combine(n_pad, h, hln, nbp, bp['eps'].reshape(1, 1),
                              bp['W1'], r1(bp['b1']), bp['W2'], r1(bp['b2']),
                              *nxt)

    n_pad = _padded(10000)
    st = _pos_stats(pos)
    bp0, bp1 = p['stages'][3]
    h, hln = _pe_add_ln(n_pad, pos, st, p['pos1_W'], r1(p['pos1_b']),
                        p['pos2_W'], r1(p['pos2_b']), h,
                        r1(bp0['g']), r1(bp0['b']))
    for bi, bp in enumerate((bp0, bp1)):
        nbp = _spmm_kernel(10000)(dst_a, src_a, hln)
        nxt = (r1(bp1['g']), r1(bp1['b'])) if bi == 0 else (None, None)
        h, hln = _combine(n_pad, h, hln, nbp, bp['eps'].reshape(1, 1),
                          bp['W1'], r1(bp['b1']), bp['W2'], r1(bp['b2']),
                          *nxt)

    return _final(h, r1(p['out_norm_g']), r1(p['out_norm_b']),
                  p['out_W'], r1(p['out_b']))


# speedup vs baseline: 22.3393x; 22.3393x over previous
import functools

import jax
import jax.numpy as jnp
from jax import lax
from jax.experimental import pallas as pl
from jax.experimental.pallas import tpu as pltpu
from jax.experimental.pallas import tpu_sc as plsc

HID = 128
EDG = 320000
NC, NS, LANES = 2, 16, 16
NW = NC * NS
NPAD = 10112
KC = 64
BR, GR = 632, 16


def _silu(x):
    return x * jax.nn.sigmoid(x)


def _ln_rows(x, g, b):
    m = jnp.mean(x, axis=-1, keepdims=True)
    v = jnp.mean((x - m) ** 2, axis=-1, keepdims=True)
    return (x - m) / jnp.sqrt(v + 1e-5) * g + b


def _dt(x, w):
    return lax.dot_general(x, w, (((1,), (1,)), ((), ())),
                           preferred_element_type=jnp.float32)


def _fill_zero(ref, nrows):
    zv = jnp.zeros((LANES,), jnp.float32)
    for r in range(nrows):
        zr = ref.at[r]
        for j in range(HID // LANES):
            zr[pl.ds(j * LANES, LANES)] = zv


ACC_R = 2560
TRASH = 2528


@functools.lru_cache(maxsize=None)
def _spmm_kernel():
    rpt = ACC_R // NS
    ept = EDG // NS
    cap = ept + 2 * KC
    mesh = plsc.VectorSubcoreMesh(core_axis_name="c", subcore_axis_name="s")

    @functools.partial(
        pl.kernel,
        out_type=jax.ShapeDtypeStruct((2 * NC, ACC_R, HID), jnp.float32),
        mesh=mesh,
        compiler_params=pltpu.CompilerParams(needs_layout_passes=False),
        scratch_types=[
            pltpu.VMEM_SHARED((ACC_R, HID), jnp.float32),
            pltpu.VMEM((64, HID), jnp.float32),
            pltpu.VMEM((ept,), jnp.int32),
            pltpu.VMEM((ept,), jnp.int32),
            pltpu.VMEM((2 * LANES,), jnp.int32),
            pltpu.VMEM((cap,), jnp.int32),
            pltpu.VMEM((cap,), jnp.int32),
            pltpu.VMEM((KC,), jnp.int32),
            pltpu.VMEM((KC,), jnp.int32),
            pltpu.VMEM((KC, HID), jnp.float32),
            pltpu.SemaphoreType.DMA,
        ],
    )
    def spmm(edge_hbm, h_hbm, pv_hbm, out_hbm, acc, zbig,
             dall, sall, prm, gpk, spk, gbuf, sbuf, rows, sem):
        cid = lax.axis_index("c")
        sid = lax.axis_index("s")
        r0 = pl.multiple_of(sid * rpt, 8)
        pltpu.sync_copy(pv_hbm, prm)
        vn = prm[pl.ds(0, LANES)]
        vq = prm[pl.ds(LANES, LANES)]
        e0 = sid * ept
        pltpu.sync_copy(edge_hbm.at[pl.ds(e0, ept)], dall)
        pltpu.sync_copy(edge_hbm.at[pl.ds(EDG + e0, ept)], sall)

        lanes = lax.iota(jnp.int32, LANES)
        _fill_zero(zbig, 64)

        for p in range(2):
            qn = cid * 2 + p
            vbase = vq * qn

            def compact(i, cnt):
                o = i * LANES
                vd = dall[pl.ds(o, LANES)]
                vs = sall[pl.ds(o, LANES)]
                ok = ((vd >= vbase) & (vd < vbase + vq) & (vd < vn) &
                      (vs < vn))
                pos = cnt - 1 + plsc.cumsum(jnp.where(ok, 1, 0))
                plsc.store_scatter(gpk, [pos], vs, mask=ok)
                plsc.store_scatter(spk, [pos], vd - vbase, mask=ok)
                pc = plsc.all_reduce_population_count(ok)
                return cnt + pc[0]

            cnt = lax.fori_loop(0, ept // LANES, compact, 0)
            for t in range(KC // LANES):
                plsc.store_scatter(gpk, [cnt + t * LANES + lanes], lanes)
                plsc.store_scatter(spk, [cnt + t * LANES + lanes],
                                   TRASH + lanes)

            for i in range(rpt // 64):
                pltpu.sync_copy(zbig, acc.at[pl.ds(r0 + 64 * i, 64)])
            if rpt % 64:
                pltpu.sync_copy(
                    zbig.at[pl.ds(0, rpt % 64)],
                    acc.at[pl.ds(r0 + (rpt // 64) * 64, rpt % 64)])
            plsc.subcore_barrier()

            def chunk(c, carry):
                o = c * KC
                for j in range(KC // LANES):
                    gbuf[pl.ds(j * LANES, LANES)] = gpk[pl.ds(
                        o + j * LANES, LANES)]
                    sbuf[pl.ds(j * LANES, LANES)] = spk[pl.ds(
                        o + j * LANES, LANES)]
                g = pltpu.async_copy(h_hbm.at[gbuf], rows, sem)
                g.wait()
                pltpu.sync_copy(rows, acc.at[sbuf], add=True)
                return carry

            nch = lax.shift_right_logical(cnt + (KC - 1), 6)
            lax.fori_loop(0, nch, chunk, 0)
            plsc.subcore_barrier()
            pltpu.sync_copy(acc.at[pl.ds(r0, rpt)],
                            out_hbm.at[qn, pl.ds(r0, rpt)])

    return spmm


@functools.lru_cache(maxsize=None)
def _unpool_kernel(m, m_pad, n):
    rpt = 320
    mesh = plsc.VectorSubcoreMesh(core_axis_name="c", subcore_axis_name="s")

    @functools.partial(
        pl.kernel,
        out_type=jax.ShapeDtypeStruct((NPAD, HID), jnp.float32),
        mesh=mesh,
        compiler_params=pltpu.CompilerParams(needs_layout_passes=False),
        scratch_types=[
            pltpu.VMEM((64, HID), jnp.float32),
            pltpu.VMEM((m_pad,), jnp.int32),
            pltpu.VMEM((KC,), jnp.int32),
            pltpu.VMEM((KC,), jnp.int32),
            pltpu.VMEM((KC, HID), jnp.float32),
            pltpu.SemaphoreType.DMA,
        ],
    )
    def unpool(rows_hbm, idx_hbm, out_hbm, zbig, kall, gbuf, sbuf,
               rows, sem):
        cid = lax.axis_index("c")
        sid = lax.axis_index("s")
        wid = sid * NC + cid
        lo = pl.multiple_of(wid * rpt, 64)
        pltpu.sync_copy(idx_hbm, kall)
        _fill_zero(zbig, 64)

        lanes = lax.iota(jnp.int32, LANES)
        vlo = 0 * lanes + lo
        vhi = jnp.minimum(vlo + rpt, NPAD)

        def scan(i, c):
            v = kall[pl.ds(i * LANES, LANES)]
            cl = plsc.all_reduce_population_count(v < vlo)
            ch = plsc.all_reduce_population_count(v < vhi)
            return (c[0] + cl[0], c[1] + ch[0])

        jlo, jhi = lax.fori_loop(0, m_pad // LANES, scan, (0, 0))

        nzb = (NPAD - NW * rpt + rpt) // 64
        for b in range(nzb):
            pltpu.sync_copy(zbig, out_hbm.at[pl.ds(lo + 64 * b, 64)])
        for b in range(nzb, rpt // 64):
            bb = b

            @pl.when(wid < NW - 1)
            def _():
                pltpu.sync_copy(zbig, out_hbm.at[pl.ds(lo + 64 * bb, 64)])

        def chunk(c, carry):
            o = jlo + c * KC
            for j in range(KC // LANES):
                src = o + j * LANES + lanes
                inb = src < jhi
                srcc = jnp.where(inb, src, m_pad - 1)
                kv = plsc.load_gather(kall, [srcc])
                gbuf[pl.ds(j * LANES, LANES)] = jnp.where(srcc < m, srcc,
                                                          m - 1)
                sbuf[pl.ds(j * LANES, LANES)] = jnp.where(inb, kv, n)
            g = pltpu.async_copy(rows_hbm.at[gbuf], rows, sem)
            g.wait()
            pltpu.sync_copy(rows, out_hbm.at[sbuf])
            return carry

        nch = lax.shift_right_logical(jhi - jlo + (KC - 1), 6)
        lax.fori_loop(0, nch, chunk, 0)

    return unpool


def _lat1(z, w, b):
    def body(z_ref, w_ref, b_ref, o_ref):
        r = _silu(_dt(z_ref[...], w_ref[...]) + b_ref[...])
        o_ref[...] = jnp.broadcast_to(r, (8, 2 * HID))

    return pl.pallas_call(
        body, out_shape=jax.ShapeDtypeStruct((8, 2 * HID), jnp.float32),
    )(z, w, b)


def _lat2(w, h1, b):
    BL, G = 6400, 25

    def body(w_ref, h_ref, b_ref, o_ref):
        o_ref[...] = _dt(h_ref[...], w_ref[...]) + b_ref[...]

    return pl.pallas_call(
        body,
        grid=(G,),
        in_specs=[
            pl.BlockSpec((BL, 2 * HID), lambda i: (i, 0)),
            pl.BlockSpec((8, 2 * HID), lambda i: (0, 0)),
            pl.BlockSpec((8, BL), lambda i: (0, i)),
        ],
        out_specs=pl.BlockSpec((8, BL), lambda i: (0, i)),
        out_shape=jax.ShapeDtypeStruct((8, BL * G), jnp.float32),
    )(w, h1, b)


_wspec = pl.BlockSpec((HID, HID), lambda i: (0, 0))
_vspec = pl.BlockSpec((1, HID), lambda i: (0, 0))
_rspec = pl.BlockSpec((BR, HID), lambda i: (i, 0))


def _up_ln(hup, w, b, g, bl):
    def body(p_ref, w_ref, b_ref, g_ref, bl_ref, h_ref, l_ref):
        h = _dt(p_ref[...], w_ref[...]) + b_ref[...]
        h_ref[...] = h
        l_ref[...] = _ln_rows(h, g_ref[...], bl_ref[...])

    return pl.pallas_call(
        body,
        grid=(GR,),
        in_specs=[_rspec, _wspec, _vspec, _vspec, _vspec],
        out_specs=[_rspec, _rspec],
        out_shape=[jax.ShapeDtypeStruct((NPAD, HID), jnp.float32)] * 2,
    )(hup, w, b, g, bl)


def _combine(n, h, hln, pp, eps, w1, b1, w2, b2, g=None, bl=None):
    out_ln = g is not None
    q = -(-n // (4 * BR))
    lbmax = ACC_R // BR - 1

    def nb_map(i):
        qn = jnp.minimum(i // q, 3)
        return (qn, jnp.minimum(i - qn * q, lbmax), 0)

    def body(h_ref, l_ref, p_ref, e_ref, w1_ref, b1_ref, w2_ref, b2_ref,
             *rest):
        x = (1.0 + e_ref[0, 0]) * l_ref[...] + p_ref[0]
        t = _silu(_dt(x, w1_ref[...]) + b1_ref[...])
        t = _dt(t, w2_ref[...]) + b2_ref[...]
        ho = h_ref[...] + t
        if out_ln:
            g_ref, bl_ref, ho_ref, lo_ref = rest
            ho_ref[...] = ho
            lo_ref[...] = _ln_rows(ho, g_ref[...], bl_ref[...])
        else:
            rest[0][...] = ho

    espec = pl.BlockSpec((1, 1), lambda i: (0, 0))
    in_specs = [_rspec, _rspec,
                pl.BlockSpec((1, BR, HID), nb_map),
                espec, _wspec, _vspec, _wspec, _vspec]
    args = [h, hln, pp, eps, w1, b1, w2, b2]
    if out_ln:
        in_specs += [_vspec, _vspec]
        args += [g, bl]
        out_specs = [_rspec, _rspec]
        out_shape = [jax.ShapeDtypeStruct((NPAD, HID), jnp.float32)] * 2
    else:
        out_specs = [_rspec]
        out_shape = [jax.ShapeDtypeStruct((NPAD, HID), jnp.float32)]
    res = pl.pallas_call(body, grid=(GR,), in_specs=in_specs,
                         out_specs=out_specs, out_shape=out_shape)(*args)
    return res if out_ln else (res[0], None)


def _pos_stats(pos):
    def body(p_ref, o_ref):
        x = p_ref[...]
        mean = jnp.sum(x, axis=0, keepdims=True) / 10000.0
        ss = jnp.sum((x - mean) ** 2, axis=0, keepdims=True)
        sd = jnp.sqrt(ss / 9999.0)
        o_ref[...] = jnp.concatenate([mean, sd], axis=0)

    return pl.pallas_call(
        body, out_shape=jax.ShapeDtypeStruct((2, 3), jnp.float32))(pos)


def _pe_add_ln(pos, st, w1, b1, w2, b2, h, g, bl):
    def body(p_ref, st_ref, w1_ref, b1_ref, w2_ref, b2_ref, h_ref, g_ref,
             bl_ref, ho_ref, lo_ref):
        s = st_ref[...]
        pn = (p_ref[...] - s[0:1, :]) / (s[1:2, :] + 1e-8)
        t = _silu(_dt(pn, w1_ref[...]) + b1_ref[...])
        pe = _dt(t, w2_ref[...]) + b2_ref[...]
        ho = h_ref[...] + pe
        ho_ref[...] = ho
        lo_ref[...] = _ln_rows(ho, g_ref[...], bl_ref[...])

    return pl.pallas_call(
        body,
        grid=(GR,),
        in_specs=[pl.BlockSpec((BR, 3), lambda i: (i, 0)),
                  pl.BlockSpec((2, 3), lambda i: (0, 0)),
                  pl.BlockSpec((HID, 3), lambda i: (0, 0)), _vspec,
                  _wspec, _vspec, _rspec, _vspec, _vspec],
        out_specs=[_rspec, _rspec],
        out_shape=[jax.ShapeDtypeStruct((NPAD, HID), jnp.float32)] * 2,
    )(pos, st, w1, b1, w2, b2, h, g, bl)


def _final(h, g, bl, w, b):
    BL, G = 1000, 10

    def body(h_ref, g_ref, bl_ref, w_ref, b_ref, o_ref):
        x = _silu(_ln_rows(h_ref[...], g_ref[...], bl_ref[...]))
        o_ref[...] = _dt(x, w_ref[...]) + b_ref[...]

    vspec = pl.BlockSpec((1, HID), lambda i: (0, 0))
    rspec = pl.BlockSpec((BL, HID), lambda i: (i, 0))
    return pl.pallas_call(
        body,
        grid=(G,),
        in_specs=[rspec, vspec, vspec,
                  pl.BlockSpec((HID, HID), lambda i: (0, 0)), vspec],
        out_specs=rspec,
        out_shape=jax.ShapeDtypeStruct((BL * G, HID), jnp.float32),
    )(h, g, bl, w, b)


def kernel(z, pos, edge_index, keep_idx0, keep_idx1, keep_idx2, params):
    p = params
    r1 = lambda a: a.reshape(1, -1)
    ef = edge_index.reshape(-1)

    def spmm(hln, n):
        pv = jnp.concatenate([
            jnp.full((LANES,), n, jnp.int32),
            jnp.full((LANES,), -(-n // (4 * BR)) * BR, jnp.int32)])
        return _spmm_kernel()(ef, hln, pv)

    h1 = _lat1(z, p['lp1_W'], r1(p['lp1_b']))
    b2 = jnp.broadcast_to(p['lp2_b'].reshape(1, -1), (8, 160000))
    h = _lat2(p['lp2_W'], h1, b2)[0].reshape(1250, HID)

    keeps = [keep_idx2, keep_idx1, keep_idx0]
    ms = [1250, 2500, 5000]
    ns = [2500, 5000, 10000]
    for d in range(3):
        m, n = ms[d], ns[d]
        m_pad = -(-m // KC) * KC
        kp = jnp.concatenate(
            [keeps[d], jnp.full((m_pad - m,), n, jnp.int32)]) \
            if m_pad > m else keeps[d]
        hup = _unpool_kernel(m, m_pad, n)(h, kp)
        bp0, bp1 = p['stages'][d]
        h, hln = _up_ln(hup, p['up_W'][d], r1(p['up_b'][d]),
                        r1(bp0['g']), r1(bp0['b']))
        for bi, bp in enumerate((bp0, bp1)):
            nbp = spmm(hln, n)
            nxt = (r1(bp1['g']), r1(bp1['b'])) if bi == 0 else (None, None)
            h, hln = _combine(n, h, hln, nbp, bp['eps'].reshape(1, 1),
                              bp['W1'], r1(bp['b1']), bp['W2'], r1(bp['b2']),
                              *nxt)

    st = _pos_stats(pos)
    bp0, bp1 = p['stages'][3]
    h, hln = _pe_add_ln(pos, st, p['pos1_W'], r1(p['pos1_b']),
                        p['pos2_W'], r1(p['pos2_b']), h,
                        r1(bp0['g']), r1(bp0['b']))
    for bi, bp in enumerate((bp0, bp1)):
        nbp = spmm(hln, 10000)
        nxt = (r1(bp1['g']), r1(bp1['b'])) if bi == 0 else (None, None)
        h, hln = _combine(10000, h, hln, nbp, bp['eps'].reshape(1, 1),
                          bp['W1'], r1(bp['b1']), bp['W2'], r1(bp['b2']),
                          *nxt)

    return _final(h, r1(p['out_norm_g']), r1(p['out_norm_b']),
                  p['out_W'], r1(p['out_b']))


# speedup vs baseline: 25.7811x; 1.1541x over previous
import functools

import jax
import jax.numpy as jnp
from jax import lax
from jax.experimental import pallas as pl
from jax.experimental.pallas import tpu as pltpu
from jax.experimental.pallas import tpu_sc as plsc

HID = 128
EDG = 320000
NC, NS, LANES = 2, 16, 16
NW = NC * NS
NPAD = 10112
KC = 64
BR, GR = 632, 16


def _silu(x):
    return x * jax.nn.sigmoid(x)


def _ln_rows(x, g, b):
    m = jnp.mean(x, axis=-1, keepdims=True)
    v = jnp.mean((x - m) ** 2, axis=-1, keepdims=True)
    return (x - m) / jnp.sqrt(v + 1e-5) * g + b


def _dt(x, w):
    return lax.dot_general(x, w, (((1,), (1,)), ((), ())),
                           preferred_element_type=jnp.float32)


def _fill_zero(ref, nrows):
    zv = jnp.zeros((LANES,), jnp.float32)
    for r in range(nrows):
        zr = ref.at[r]
        for j in range(HID // LANES):
            zr[pl.ds(j * LANES, LANES)] = zv


ACC_R = 2560
TRASH = 2528


@functools.lru_cache(maxsize=None)
def _spmm_kernel():
    rpt = ACC_R // NS
    ept = EDG // NS
    cap = ept + 2 * KC
    mesh = plsc.VectorSubcoreMesh(core_axis_name="c", subcore_axis_name="s")

    @functools.partial(
        pl.kernel,
        out_type=jax.ShapeDtypeStruct((2 * NC, ACC_R, HID), jnp.float32),
        mesh=mesh,
        compiler_params=pltpu.CompilerParams(needs_layout_passes=False),
        scratch_types=[
            pltpu.VMEM_SHARED((ACC_R, HID), jnp.float32),
            pltpu.VMEM((64, HID), jnp.float32),
            pltpu.VMEM((ept,), jnp.int32),
            pltpu.VMEM((ept,), jnp.int32),
            pltpu.VMEM((2 * LANES,), jnp.int32),
            pltpu.VMEM((cap,), jnp.int32),
            pltpu.VMEM((cap,), jnp.int32),
            pltpu.VMEM((KC,), jnp.int32),
            pltpu.VMEM((KC,), jnp.int32),
            pltpu.VMEM((KC, HID), jnp.float32),
            pltpu.VMEM((KC,), jnp.int32),
            pltpu.VMEM((KC,), jnp.int32),
            pltpu.VMEM((KC, HID), jnp.float32),
            pltpu.SemaphoreType.DMA,
            pltpu.SemaphoreType.DMA,
        ],
    )
    def spmm(edge_hbm, h_hbm, pv_hbm, out_hbm, acc, zbig,
             dall, sall, prm, gpk, spk, gbuf, sbuf, rows, gbuf1, sbuf1,
             rows1, sem, sem1):
        cid = lax.axis_index("c")
        sid = lax.axis_index("s")
        r0 = pl.multiple_of(sid * rpt, 8)
        pltpu.sync_copy(pv_hbm, prm)
        vn = prm[pl.ds(0, LANES)]
        vq = prm[pl.ds(LANES, LANES)]
        e0 = sid * ept
        pltpu.sync_copy(edge_hbm.at[pl.ds(e0, ept)], dall)
        pltpu.sync_copy(edge_hbm.at[pl.ds(EDG + e0, ept)], sall)

        lanes = lax.iota(jnp.int32, LANES)
        _fill_zero(zbig, 64)

        for p in range(2):
            qn = cid * 2 + p
            vbase = vq * qn

            def compact(i, cnt):
                o = i * LANES
                vd = dall[pl.ds(o, LANES)]
                vs = sall[pl.ds(o, LANES)]
                ok = ((vd >= vbase) & (vd < vbase + vq) & (vd < vn) &
                      (vs < vn))
                pos = cnt - 1 + plsc.cumsum(jnp.where(ok, 1, 0))
                plsc.store_scatter(gpk, [pos], vs, mask=ok)
                plsc.store_scatter(spk, [pos], vd - vbase, mask=ok)
                pc = plsc.all_reduce_population_count(ok)
                return cnt + pc[0]

            cnt = lax.fori_loop(0, ept // LANES, compact, 0)
            for t in range(2 * KC // LANES):
                plsc.store_scatter(gpk, [cnt + t * LANES + lanes], lanes)
                plsc.store_scatter(spk, [cnt + t * LANES + lanes],
                                   TRASH + lanes)

            for i in range(rpt // 64):
                pltpu.sync_copy(zbig, acc.at[pl.ds(r0 + 64 * i, 64)])
            if rpt % 64:
                pltpu.sync_copy(
                    zbig.at[pl.ds(0, rpt % 64)],
                    acc.at[pl.ds(r0 + (rpt // 64) * 64, rpt % 64)])
            plsc.subcore_barrier()

            def pair(c, carry):
                o = c * (2 * KC)
                for j in range(KC // LANES):
                    gbuf[pl.ds(j * LANES, LANES)] = gpk[pl.ds(
                        o + j * LANES, LANES)]
                    sbuf[pl.ds(j * LANES, LANES)] = spk[pl.ds(
                        o + j * LANES, LANES)]
                g0 = pltpu.async_copy(h_hbm.at[gbuf], rows, sem)
                for j in range(KC // LANES):
                    gbuf1[pl.ds(j * LANES, LANES)] = gpk[pl.ds(
                        o + KC + j * LANES, LANES)]
                    sbuf1[pl.ds(j * LANES, LANES)] = spk[pl.ds(
                        o + KC + j * LANES, LANES)]
                g1 = pltpu.async_copy(h_hbm.at[gbuf1], rows1, sem1)
                g0.wait()
                pltpu.sync_copy(rows, acc.at[sbuf], add=True)
                g1.wait()
                pltpu.sync_copy(rows1, acc.at[sbuf1], add=True)
                return carry

            npr = lax.shift_right_logical(cnt + (2 * KC - 1), 7)
            lax.fori_loop(0, npr, pair, 0)
            plsc.subcore_barrier()
            pltpu.sync_copy(acc.at[pl.ds(r0, rpt)],
                            out_hbm.at[qn, pl.ds(r0, rpt)])

    return spmm


@functools.lru_cache(maxsize=None)
def _unpool_kernel(m, m_pad, n):
    rpt = 320
    mesh = plsc.VectorSubcoreMesh(core_axis_name="c", subcore_axis_name="s")

    @functools.partial(
        pl.kernel,
        out_type=jax.ShapeDtypeStruct((NPAD, HID), jnp.float32),
        mesh=mesh,
        compiler_params=pltpu.CompilerParams(needs_layout_passes=False),
        scratch_types=[
            pltpu.VMEM((64, HID), jnp.float32),
            pltpu.VMEM((m_pad,), jnp.int32),
            pltpu.VMEM((KC,), jnp.int32),
            pltpu.VMEM((KC,), jnp.int32),
            pltpu.VMEM((KC, HID), jnp.float32),
            pltpu.SemaphoreType.DMA,
        ],
    )
    def unpool(rows_hbm, idx_hbm, out_hbm, zbig, kall, gbuf, sbuf,
               rows, sem):
        cid = lax.axis_index("c")
        sid = lax.axis_index("s")
        wid = sid * NC + cid
        lo = pl.multiple_of(wid * rpt, 64)
        pltpu.sync_copy(idx_hbm, kall)
        _fill_zero(zbig, 64)

        lanes = lax.iota(jnp.int32, LANES)
        vlo = 0 * lanes + lo
        vhi = jnp.minimum(vlo + rpt, NPAD)

        def scan(i, c):
            v = kall[pl.ds(i * LANES, LANES)]
            cl = plsc.all_reduce_population_count(v < vlo)
            ch = plsc.all_reduce_population_count(v < vhi)
            return (c[0] + cl[0], c[1] + ch[0])

        jlo, jhi = lax.fori_loop(0, m_pad // LANES, scan, (0, 0))

        nzb = (NPAD - NW * rpt + rpt) // 64
        for b in range(nzb):
            pltpu.sync_copy(zbig, out_hbm.at[pl.ds(lo + 64 * b, 64)])
        for b in range(nzb, rpt // 64):
            bb = b

            @pl.when(wid < NW - 1)
            def _():
                pltpu.sync_copy(zbig, out_hbm.at[pl.ds(lo + 64 * bb, 64)])

        def chunk(c, carry):
            o = jlo + c * KC
            for j in range(KC // LANES):
                src = o + j * LANES + lanes
                inb = src < jhi
                srcc = jnp.where(inb, src, m_pad - 1)
                kv = plsc.load_gather(kall, [srcc])
                gbuf[pl.ds(j * LANES, LANES)] = jnp.where(srcc < m, srcc,
                                                          m - 1)
                sbuf[pl.ds(j * LANES, LANES)] = jnp.where(inb, kv, n)
            g = pltpu.async_copy(rows_hbm.at[gbuf], rows, sem)
            g.wait()
            pltpu.sync_copy(rows, out_hbm.at[sbuf])
            return carry

        nch = lax.shift_right_logical(jhi - jlo + (KC - 1), 6)
        lax.fori_loop(0, nch, chunk, 0)

    return unpool


def _lat1(z, w, b):
    def body(z_ref, w_ref, b_ref, o_ref):
        r = _silu(_dt(z_ref[...], w_ref[...]) + b_ref[...])
        o_ref[...] = jnp.broadcast_to(r, (8, 2 * HID))

    return pl.pallas_call(
        body, out_shape=jax.ShapeDtypeStruct((8, 2 * HID), jnp.float32),
    )(z, w, b)


def _lat2(w, h1, b):
    BL, G = 6400, 25

    def body(w_ref, h_ref, b_ref, o_ref):
        o_ref[...] = _dt(h_ref[...], w_ref[...]) + b_ref[...]

    return pl.pallas_call(
        body,
        grid=(G,),
        in_specs=[
            pl.BlockSpec((BL, 2 * HID), lambda i: (i, 0)),
            pl.BlockSpec((8, 2 * HID), lambda i: (0, 0)),
            pl.BlockSpec((8, BL), lambda i: (0, i)),
        ],
        out_specs=pl.BlockSpec((8, BL), lambda i: (0, i)),
        out_shape=jax.ShapeDtypeStruct((8, BL * G), jnp.float32),
    )(w, h1, b)


_wspec = pl.BlockSpec((HID, HID), lambda i: (0, 0))
_vspec = pl.BlockSpec((1, HID), lambda i: (0, 0))
_rspec = pl.BlockSpec((BR, HID), lambda i: (i, 0))


def _up_ln(hup, w, b, g, bl):
    def body(p_ref, w_ref, b_ref, g_ref, bl_ref, h_ref, l_ref):
        h = _dt(p_ref[...], w_ref[...]) + b_ref[...]
        h_ref[...] = h
        l_ref[...] = _ln_rows(h, g_ref[...], bl_ref[...])

    return pl.pallas_call(
        body,
        grid=(GR,),
        in_specs=[_rspec, _wspec, _vspec, _vspec, _vspec],
        out_specs=[_rspec, _rspec],
        out_shape=[jax.ShapeDtypeStruct((NPAD, HID), jnp.float32)] * 2,
    )(hup, w, b, g, bl)


def _combine(n, h, hln, pp, eps, w1, b1, w2, b2, g=None, bl=None):
    out_ln = g is not None
    q = -(-n // (4 * BR))
    lbmax = ACC_R // BR - 1

    def nb_map(i):
        qn = jnp.minimum(i // q, 3)
        return (qn, jnp.minimum(i - qn * q, lbmax), 0)

    def body(h_ref, l_ref, p_ref, e_ref, w1_ref, b1_ref, w2_ref, b2_ref,
             *rest):
        x = (1.0 + e_ref[0, 0]) * l_ref[...] + p_ref[0]
        t = _silu(_dt(x, w1_ref[...]) + b1_ref[...])
        t = _dt(t, w2_ref[...]) + b2_ref[...]
        ho = h_ref[...] + t
        if out_ln:
            g_ref, bl_ref, ho_ref, lo_ref = rest
            ho_ref[...] = ho
            lo_ref[...] = _ln_rows(ho, g_ref[...], bl_ref[...])
        else:
            rest[0][...] = ho

    espec = pl.BlockSpec((1, 1), lambda i: (0, 0))
    in_specs = [_rspec, _rspec,
                pl.BlockSpec((1, BR, HID), nb_map),
                espec, _wspec, _vspec, _wspec, _vspec]
    args = [h, hln, pp, eps, w1, b1, w2, b2]
    if out_ln:
        in_specs += [_vspec, _vspec]
        args += [g, bl]
        out_specs = [_rspec, _rspec]
        out_shape = [jax.ShapeDtypeStruct((NPAD, HID), jnp.float32)] * 2
    else:
        out_specs = [_rspec]
        out_shape = [jax.ShapeDtypeStruct((NPAD, HID), jnp.float32)]
    res = pl.pallas_call(body, grid=(GR,), in_specs=in_specs,
                         out_specs=out_specs, out_shape=out_shape)(*args)
    return res if out_ln else (res[0], None)


def _pos_stats(pos):
    def body(p_ref, o_ref):
        x = p_ref[...]
        mean = jnp.sum(x, axis=0, keepdims=True) / 10000.0
        ss = jnp.sum((x - mean) ** 2, axis=0, keepdims=True)
        sd = jnp.sqrt(ss / 9999.0)
        o_ref[...] = jnp.concatenate([mean, sd], axis=0)

    return pl.pallas_call(
        body, out_shape=jax.ShapeDtypeStruct((2, 3), jnp.float32))(pos)


def _pe_add_ln(pos, st, w1, b1, w2, b2, h, g, bl):
    def body(p_ref, st_ref, w1_ref, b1_ref, w2_ref, b2_ref, h_ref, g_ref,
             bl_ref, ho_ref, lo_ref):
        s = st_ref[...]
        pn = (p_ref[...] - s[0:1, :]) / (s[1:2, :] + 1e-8)
        t = _silu(_dt(pn, w1_ref[...]) + b1_ref[...])
        pe = _dt(t, w2_ref[...]) + b2_ref[...]
        ho = h_ref[...] + pe
        ho_ref[...] = ho
        lo_ref[...] = _ln_rows(ho, g_ref[...], bl_ref[...])

    return pl.pallas_call(
        body,
        grid=(GR,),
        in_specs=[pl.BlockSpec((BR, 3), lambda i: (i, 0)),
                  pl.BlockSpec((2, 3), lambda i: (0, 0)),
                  pl.BlockSpec((HID, 3), lambda i: (0, 0)), _vspec,
                  _wspec, _vspec, _rspec, _vspec, _vspec],
        out_specs=[_rspec, _rspec],
        out_shape=[jax.ShapeDtypeStruct((NPAD, HID), jnp.float32)] * 2,
    )(pos, st, w1, b1, w2, b2, h, g, bl)


def _final(h, g, bl, w, b):
    BL, G = 1000, 10

    def body(h_ref, g_ref, bl_ref, w_ref, b_ref, o_ref):
        x = _silu(_ln_rows(h_ref[...], g_ref[...], bl_ref[...]))
        o_ref[...] = _dt(x, w_ref[...]) + b_ref[...]

    vspec = pl.BlockSpec((1, HID), lambda i: (0, 0))
    rspec = pl.BlockSpec((BL, HID), lambda i: (i, 0))
    return pl.pallas_call(
        body,
        grid=(G,),
        in_specs=[rspec, vspec, vspec,
                  pl.BlockSpec((HID, HID), lambda i: (0, 0)), vspec],
        out_specs=rspec,
        out_shape=jax.ShapeDtypeStruct((BL * G, HID), jnp.float32),
    )(h, g, bl, w, b)


def kernel(z, pos, edge_index, keep_idx0, keep_idx1, keep_idx2, params):
    p = params
    r1 = lambda a: a.reshape(1, -1)
    ef = edge_index.reshape(-1)

    def spmm(hln, n):
        pv = jnp.concatenate([
            jnp.full((LANES,), n, jnp.int32),
            jnp.full((LANES,), -(-n // (4 * BR)) * BR, jnp.int32)])
        return _spmm_kernel()(ef, hln, pv)

    h1 = _lat1(z, p['lp1_W'], r1(p['lp1_b']))
    b2 = jnp.broadcast_to(p['lp2_b'].reshape(1, -1), (8, 160000))
    h = _lat2(p['lp2_W'], h1, b2)[0].reshape(1250, HID)

    keeps = [keep_idx2, keep_idx1, keep_idx0]
    ms = [1250, 2500, 5000]
    ns = [2500, 5000, 10000]
    for d in range(3):
        m, n = ms[d], ns[d]
        m_pad = -(-m // KC) * KC
        kp = jnp.concatenate(
            [keeps[d], jnp.full((m_pad - m,), n, jnp.int32)]) \
            if m_pad > m else keeps[d]
        hup = _unpool_kernel(m, m_pad, n)(h, kp)
        bp0, bp1 = p['stages'][d]
        h, hln = _up_ln(hup, p['up_W'][d], r1(p['up_b'][d]),
                        r1(bp0['g']), r1(bp0['b']))
        for bi, bp in enumerate((bp0, bp1)):
            nbp = spmm(hln, n)
            nxt = (r1(bp1['g']), r1(bp1['b'])) if bi == 0 else (None, None)
            h, hln = _combine(n, h, hln, nbp, bp['eps'].reshape(1, 1),
                              bp['W1'], r1(bp['b1']), bp['W2'], r1(bp['b2']),
                              *nxt)

    st = _pos_stats(pos)
    bp0, bp1 = p['stages'][3]
    h, hln = _pe_add_ln(pos, st, p['pos1_W'], r1(p['pos1_b']),
                        p['pos2_W'], r1(p['pos2_b']), h,
                        r1(bp0['g']), r1(bp0['b']))
    for bi, bp in enumerate((bp0, bp1)):
        nbp = spmm(hln, 10000)
        nxt = (r1(bp1['g']), r1(bp1['b'])) if bi == 0 else (None, None)
        h, hln = _combine(10000, h, hln, nbp, bp['eps'].reshape(1, 1),
                          bp['W1'], r1(bp['b1']), bp['W2'], r1(bp['b2']),
                          *nxt)

    return _final(h, r1(p['out_norm_g']), r1(p['out_norm_b']),
                  p['out_W'], r1(p['out_b']))
